# baseline (device time: 21249 ns/iter reference)
import jax
import jax.numpy as jnp
from jax import lax
from jax.experimental import pallas as pl
from jax.experimental.pallas import tpu as pltpu

A_ROWS = 576
FWD_ROWS = 448
FWD_SIZES = [64, 64, 64, 64, 64, 64, 32, 16, 16]
OVL_SIZES = [64, 32, 16, 16]
SIZES = FWD_SIZES + OVL_SIZES
OFFS = [sum(SIZES[:i]) for i in range(len(SIZES))]
C = len(SIZES)
NF = len(FWD_SIZES)
assert sum(FWD_SIZES) == FWD_ROWS and sum(SIZES) == A_ROWS


def kernel(partial, resid, gamma):
    m, d = resid.shape

    def body(part_ref, resid_ref, gamma_ref, out_ref,
             part_v, resid_v, gamma_v, mine, other_half, out_bf, recv_out,
             sa, ra, sb, rb, sem_part, sem_resid, sem_gamma):
        my_x = lax.axis_index("x")
        my_y = lax.axis_index("y")
        x_nbr = (1 - my_x, my_y)
        y_nbr = (my_x, 1 - my_y)

        def grow(c):
            lo = OFFS[c]
            return lo + my_y * A_ROWS if c < NF else lo

        barrier = pltpu.get_barrier_semaphore()
        for nbr in (x_nbr, y_nbr):
            pl.semaphore_signal(barrier, inc=1, device_id=nbr,
                                device_id_type=pl.DeviceIdType.MESH)
        pl.semaphore_wait(barrier, 2)

        gamma_cp = pltpu.make_async_copy(gamma_ref, gamma_v, sem_gamma)
        gamma_cp.start()
        resid_cp = pltpu.make_async_copy(
            resid_ref.at[pl.ds(my_y * FWD_ROWS, A_ROWS)], resid_v, sem_resid)
        resid_cp.start()
        part_cps = []
        for c in range(C):
            cp = pltpu.make_async_copy(
                part_ref.at[0, pl.ds(grow(c), SIZES[c])],
                part_v.at[pl.ds(OFFS[c], SIZES[c])],
                sem_part.at[c])
            cp.start()
            part_cps.append(cp)

        def rlo(c):
            return grow(c) - my_y * FWD_ROWS

        a_rdmas = []
        for c in range(C):
            lo, n = OFFS[c], SIZES[c]
            part_cps[c].wait()
            mine[pl.ds(lo, n), :] = part_v[pl.ds(lo, n), :].astype(jnp.bfloat16)
            r = pltpu.make_async_remote_copy(
                src_ref=mine.at[pl.ds(lo, n)],
                dst_ref=other_half.at[pl.ds(lo, n)],
                send_sem=sa.at[c], recv_sem=ra.at[c],
                device_id=x_nbr, device_id_type=pl.DeviceIdType.MESH,
            )
            r.start()
            a_rdmas.append(r)
        gamma_cp.wait()
        resid_cp.wait()

        b_rdmas = []
        for c in range(C):
            lo, n = OFFS[c], SIZES[c]
            a_rdmas[c].wait_recv()
            yv = (part_v[pl.ds(lo, n), :]
                  + other_half[pl.ds(lo, n), :].astype(jnp.float32)
                  + resid_v[pl.ds(rlo(c), n), :])
            ms = jnp.mean(yv * yv, axis=-1, keepdims=True)
            scaled = yv * lax.rsqrt(ms + 1e-6) * gamma_v[...]
            out_ref[pl.ds(grow(c), n), :] = scaled
            if c < NF:
                out_bf[pl.ds(lo, n), :] = scaled.astype(jnp.bfloat16)
                r = pltpu.make_async_remote_copy(
                    src_ref=out_bf.at[pl.ds(lo, n)],
                    dst_ref=recv_out.at[pl.ds(lo, n)],
                    send_sem=sb.at[c], recv_sem=rb.at[c],
                    device_id=y_nbr, device_id_type=pl.DeviceIdType.MESH,
                )
                r.start()
                b_rdmas.append(r)

        for c in range(NF):
            lo, n = OFFS[c], SIZES[c]
            b_rdmas[c].wait_recv()
            out_ref[pl.ds(lo + (1 - my_y) * A_ROWS, n), :] = (
                recv_out[pl.ds(lo, n), :].astype(jnp.float32))

        for c in range(C):
            a_rdmas[c].wait_send()
        for c in range(NF):
            b_rdmas[c].wait_send()

    return pl.pallas_call(
        body,
        out_shape=jax.ShapeDtypeStruct((m, d), jnp.float32),
        in_specs=[pl.BlockSpec(memory_space=pltpu.MemorySpace.HBM)] * 3,
        out_specs=pl.BlockSpec(memory_space=pltpu.VMEM),
        scratch_shapes=[
            pltpu.VMEM((A_ROWS, d), jnp.float32),
            pltpu.VMEM((A_ROWS, d), jnp.float32),
            pltpu.VMEM((1, d), jnp.float32),
            pltpu.VMEM((A_ROWS, d), jnp.bfloat16),
            pltpu.VMEM((A_ROWS, d), jnp.bfloat16),
            pltpu.VMEM((FWD_ROWS, d), jnp.bfloat16),
            pltpu.VMEM((FWD_ROWS, d), jnp.bfloat16),
            pltpu.SemaphoreType.DMA((C,)),
            pltpu.SemaphoreType.DMA((C,)),
            pltpu.SemaphoreType.DMA((NF,)),
            pltpu.SemaphoreType.DMA((NF,)),
            pltpu.SemaphoreType.DMA((C,)),
            pltpu.SemaphoreType.DMA,
            pltpu.SemaphoreType.DMA,
        ],
        compiler_params=pltpu.CompilerParams(collective_id=0),
    )(
        pltpu.with_memory_space_constraint(partial, pltpu.MemorySpace.HBM),
        pltpu.with_memory_space_constraint(resid, pltpu.MemorySpace.HBM),
        pltpu.with_memory_space_constraint(
            gamma.reshape(1, d), pltpu.MemorySpace.HBM),
    )


# device time: 16941 ns/iter; 1.2543x vs baseline; 1.2543x over previous
import jax
import jax.numpy as jnp
from jax import lax
from jax.experimental import pallas as pl
from jax.experimental.pallas import tpu as pltpu

A_ROWS = 768
FWD_ROWS = 256
FWD_SIZES = [64, 64, 64, 32, 32]
OVL_SIZES = [64, 64, 64, 64, 64, 64, 64, 64]
SIZES = FWD_SIZES + OVL_SIZES
OFFS = [sum(SIZES[:i]) for i in range(len(SIZES))]
C = len(SIZES)
NF = len(FWD_SIZES)
assert sum(FWD_SIZES) == FWD_ROWS and sum(SIZES) == A_ROWS


def kernel(partial, resid, gamma):
    m, d = resid.shape

    def body(part_ref, resid_ref, gamma_ref, out_ref,
             part_v, resid_v, gamma_v, mine, other_half, out_bf, recv_out,
             sa, ra, sb, rb, sem_part, sem_resid, sem_gamma):
        my_x = lax.axis_index("x")
        my_y = lax.axis_index("y")
        x_nbr = (1 - my_x, my_y)
        y_nbr = (my_x, 1 - my_y)

        def grow(c):
            lo = OFFS[c]
            return lo + my_y * A_ROWS if c < NF else lo

        barrier = pltpu.get_barrier_semaphore()
        for nbr in (x_nbr, y_nbr):
            pl.semaphore_signal(barrier, inc=1, device_id=nbr,
                                device_id_type=pl.DeviceIdType.MESH)
        pl.semaphore_wait(barrier, 2)

        gamma_cp = pltpu.make_async_copy(gamma_ref, gamma_v, sem_gamma)
        gamma_cp.start()
        resid_cp = pltpu.make_async_copy(
            resid_ref.at[pl.ds(my_y * FWD_ROWS, A_ROWS)], resid_v, sem_resid)
        resid_cp.start()
        part_cps = []
        for c in range(C):
            cp = pltpu.make_async_copy(
                part_ref.at[0, pl.ds(grow(c), SIZES[c])],
                part_v.at[pl.ds(OFFS[c], SIZES[c])],
                sem_part.at[c])
            cp.start()
            part_cps.append(cp)

        def rlo(c):
            return grow(c) - my_y * FWD_ROWS

        a_rdmas = []
        for c in range(C):
            lo, n = OFFS[c], SIZES[c]
            part_cps[c].wait()
            mine[pl.ds(lo, n), :] = (
                part_v[pl.ds(lo, n), :].astype(jnp.float8_e4m3fn))
            r = pltpu.make_async_remote_copy(
                src_ref=mine.at[pl.ds(lo, n)],
                dst_ref=other_half.at[pl.ds(lo, n)],
                send_sem=sa.at[c], recv_sem=ra.at[c],
                device_id=x_nbr, device_id_type=pl.DeviceIdType.MESH,
            )
            r.start()
            a_rdmas.append(r)
        gamma_cp.wait()
        resid_cp.wait()

        b_rdmas = []
        for c in range(C):
            lo, n = OFFS[c], SIZES[c]
            a_rdmas[c].wait_recv()
            yv = (part_v[pl.ds(lo, n), :]
                  + other_half[pl.ds(lo, n), :].astype(jnp.float32)
                  + resid_v[pl.ds(rlo(c), n), :])
            ms = jnp.mean(yv * yv, axis=-1, keepdims=True)
            scaled = yv * lax.rsqrt(ms + 1e-6) * gamma_v[...]
            out_ref[pl.ds(grow(c), n), :] = scaled
            if c < NF:
                out_bf[pl.ds(lo, n), :] = scaled.astype(jnp.bfloat16)
                r = pltpu.make_async_remote_copy(
                    src_ref=out_bf.at[pl.ds(lo, n)],
                    dst_ref=recv_out.at[pl.ds(lo, n)],
                    send_sem=sb.at[c], recv_sem=rb.at[c],
                    device_id=y_nbr, device_id_type=pl.DeviceIdType.MESH,
                )
                r.start()
                b_rdmas.append(r)

        for c in range(NF):
            lo, n = OFFS[c], SIZES[c]
            b_rdmas[c].wait_recv()
            out_ref[pl.ds(lo + (1 - my_y) * A_ROWS, n), :] = (
                recv_out[pl.ds(lo, n), :].astype(jnp.float32))

        for c in range(C):
            a_rdmas[c].wait_send()
        for c in range(NF):
            b_rdmas[c].wait_send()

    return pl.pallas_call(
        body,
        out_shape=jax.ShapeDtypeStruct((m, d), jnp.float32),
        in_specs=[pl.BlockSpec(memory_space=pltpu.MemorySpace.HBM)] * 3,
        out_specs=pl.BlockSpec(memory_space=pltpu.VMEM),
        scratch_shapes=[
            pltpu.VMEM((A_ROWS, d), jnp.float32),
            pltpu.VMEM((A_ROWS, d), jnp.float32),
            pltpu.VMEM((1, d), jnp.float32),
            pltpu.VMEM((A_ROWS, d), jnp.float8_e4m3fn),
            pltpu.VMEM((A_ROWS, d), jnp.float8_e4m3fn),
            pltpu.VMEM((FWD_ROWS, d), jnp.bfloat16),
            pltpu.VMEM((FWD_ROWS, d), jnp.bfloat16),
            pltpu.SemaphoreType.DMA((C,)),
            pltpu.SemaphoreType.DMA((C,)),
            pltpu.SemaphoreType.DMA((NF,)),
            pltpu.SemaphoreType.DMA((NF,)),
            pltpu.SemaphoreType.DMA((C,)),
            pltpu.SemaphoreType.DMA,
            pltpu.SemaphoreType.DMA,
        ],
        compiler_params=pltpu.CompilerParams(collective_id=0),
    )(
        pltpu.with_memory_space_constraint(partial, pltpu.MemorySpace.HBM),
        pltpu.with_memory_space_constraint(resid, pltpu.MemorySpace.HBM),
        pltpu.with_memory_space_constraint(
            gamma.reshape(1, d), pltpu.MemorySpace.HBM),
    )
